# baseline (speedup 1.0000x reference)
import functools
import jax
import jax.numpy as jnp
from jax import lax
from jax.experimental import pallas as pl
from jax.experimental.pallas import tpu as pltpu
from jax.experimental.pallas import tpu_sc as plsc

N = 10000
E = 160000
NF = 128
DIM = 32
NG = 64
STEPS = 3

NP = 10240
NP4 = NP // 4
NW = 32
CHUNK = 128
EPW = 5120
EPAD = NW * EPW
EP4 = EPAD // 4
NCH = EPW // CHUNK
ACC = 10256
ZPW = ACC // 16
OPW = NP // 16
BE4 = 512


@functools.cache
def _mesh():
    return plsc.VectorSubcoreMesh(core_axis_name="c", subcore_axis_name="s")


SEG = 4
SGC = NCH // SEG
SGR = SGC * CHUNK


def _gather_pipeline(table, idx_v, out, buf_a, buf_b, sem_g, sem_w, base,
                     seg_hook=None):
    bufs = [buf_a, buf_b]
    writes = [None, None]
    for g in range(SEG):
        buf = bufs[g % 2]
        if writes[g % 2] is not None:
            writes[g % 2].wait()
        cp = pltpu.async_copy(
            table.at[idx_v.at[pl.ds(g * SGR, SGR)]], buf, sem_g
        )
        if seg_hook is not None:
            seg_hook(g)
        cp.wait()
        writes[g % 2] = pltpu.async_copy(
            buf, out.at[pl.ds(base + g * SGR, SGR)], sem_w
        )
    writes[0].wait()
    writes[1].wait()


def _gather_body(table, idx3, out, idx_v, buf_a, buf_b, sem_g, sem_w):
    c = lax.axis_index("c")
    s = lax.axis_index("s")
    wid = s * 2 + c
    pltpu.sync_copy(idx3.at[wid], idx_v)
    _gather_pipeline(table, idx_v, out, buf_a, buf_b, sem_g, sem_w, wid * EPW)


def _sc_gather(table, idx3):
    return pl.kernel(
        _gather_body,
        out_type=jax.ShapeDtypeStruct((EPAD, DIM), jnp.float32),
        mesh=_mesh(),
        compiler_params=pltpu.CompilerParams(use_tc_tiling_on_sc=False),
        scratch_types=[
            pltpu.VMEM((EPW,), jnp.int32),
            pltpu.VMEM((SGR, DIM), jnp.float32),
            pltpu.VMEM((SGR, DIM), jnp.float32),
            pltpu.SemaphoreType.DMA,
            pltpu.SemaphoreType.DMA,
        ],
    )(table, idx3)


def _gather_count_body(table, idx3, cidx3, ones, zeros, out, cnt_out,
                       acc, idx_v, cidx_v, ones_v, buf_a, buf_b,
                       sem_g, sem_w):
    c = lax.axis_index("c")
    s = lax.axis_index("s")
    wid = s * 2 + c
    pltpu.sync_copy(zeros.at[pl.ds(s * ZPW, ZPW)], acc.at[pl.ds(s * ZPW, ZPW)])
    pltpu.sync_copy(idx3.at[wid], idx_v)
    pltpu.sync_copy(cidx3.at[wid], cidx_v)
    pltpu.sync_copy(ones, ones_v)
    plsc.subcore_barrier()

    def count_hook(g):
        def body(j, carry):
            pltpu.sync_copy(ones_v, acc.at[cidx_v.at[g * SGC + j]], add=True)
            return carry
        lax.fori_loop(0, SGC, body, 0, unroll=False)

    _gather_pipeline(table, idx_v, out, buf_a, buf_b, sem_g, sem_w,
                     wid * EPW, seg_hook=count_hook)
    plsc.subcore_barrier()
    pltpu.sync_copy(acc.at[pl.ds(s * OPW, OPW)],
                    cnt_out.at[c].at[pl.ds(s * OPW, OPW)])


def _sc_gather_count(table, idx3, cidx3, ones, zeros):
    return pl.kernel(
        _gather_count_body,
        out_type=(
            jax.ShapeDtypeStruct((EPAD, DIM), jnp.float32),
            jax.ShapeDtypeStruct((2, NP, DIM), jnp.float32),
        ),
        mesh=_mesh(),
        compiler_params=pltpu.CompilerParams(use_tc_tiling_on_sc=False),
        scratch_types=[
            pltpu.VMEM_SHARED((ACC, DIM), jnp.float32),
            pltpu.VMEM((EPW,), jnp.int32),
            pltpu.VMEM((NCH, CHUNK), jnp.int32),
            pltpu.VMEM((CHUNK, DIM), jnp.float32),
            pltpu.VMEM((SGR, DIM), jnp.float32),
            pltpu.VMEM((SGR, DIM), jnp.float32),
            pltpu.SemaphoreType.DMA,
            pltpu.SemaphoreType.DMA,
        ],
    )(table, idx3, cidx3, ones, zeros)


def _scatter_body(msg, idx3, zeros, out, acc, idx_v, buf_v):
    c = lax.axis_index("c")
    s = lax.axis_index("s")
    wid = s * 2 + c
    pltpu.sync_copy(zeros.at[pl.ds(s * ZPW, ZPW)], acc.at[pl.ds(s * ZPW, ZPW)])
    pltpu.sync_copy(idx3.at[wid], idx_v)
    plsc.subcore_barrier()

    def body(j, carry):
        pltpu.sync_copy(msg.at[pl.ds(wid * EPW + j * CHUNK, CHUNK)], buf_v)
        pltpu.sync_copy(buf_v, acc.at[idx_v.at[j]], add=True)
        return carry

    lax.fori_loop(0, NCH, body, 0, unroll=False)
    plsc.subcore_barrier()
    pltpu.sync_copy(acc.at[pl.ds(s * OPW, OPW)], out.at[c].at[pl.ds(s * OPW, OPW)])


def _sc_scatter(msg, idx3, zeros):
    return pl.kernel(
        _scatter_body,
        out_type=jax.ShapeDtypeStruct((2, NP, DIM), jnp.float32),
        mesh=_mesh(),
        compiler_params=pltpu.CompilerParams(use_tc_tiling_on_sc=False),
        scratch_types=[
            pltpu.VMEM_SHARED((ACC, DIM), jnp.float32),
            pltpu.VMEM((NCH, CHUNK), jnp.int32),
            pltpu.VMEM((CHUNK, DIM), jnp.float32),
        ],
    )(msg, idx3, zeros)


def _proj_body(x_ref, w_ref, b_ref, o_ref):
    o_ref[...] = jax.nn.relu(
        jnp.dot(x_ref[...], w_ref[...], preferred_element_type=jnp.float32)
        + b_ref[...]
    )


def _tc_proj(xq, w0bd, b0t):
    return pl.pallas_call(
        _proj_body,
        in_specs=[
            pl.BlockSpec((NP4, 4 * NF), lambda: (0, 0)),
            pl.BlockSpec((4 * NF, 128), lambda: (0, 0)),
            pl.BlockSpec((1, 128), lambda: (0, 0)),
        ],
        out_specs=pl.BlockSpec((NP4, 128), lambda: (0, 0)),
        out_shape=jax.ShapeDtypeStruct((NP4, 128), jnp.float32),
    )(xq, w0bd, b0t)


def _edge_body(ea_ref, xj_ref, w1_ref, b1_ref, w2_ref, b2_ref, r_ref, o_ref):
    hidden = jax.nn.relu(
        jnp.dot(ea_ref[...], w1_ref[...], preferred_element_type=jnp.float32)
        + b1_ref[...]
    ).astype(jnp.bfloat16)
    theta = (
        jnp.dot(hidden, w2_ref[...], preferred_element_type=jnp.float32)
        + b2_ref[...]
    )
    xjr = jnp.dot(xj_ref[...].astype(jnp.bfloat16), r_ref[...],
                  preferred_element_type=jnp.float32)
    prod = theta * xjr
    for a in range(4):
        s1 = prod[:, 1024 * a:1024 * a + 128]
        for q in range(1, 8):
            s1 = s1 + prod[:, 1024 * a + 128 * q:1024 * a + 128 * (q + 1)]
        m = s1[:, 0:DIM]
        for c in range(1, 4):
            m = m + s1[:, DIM * c:DIM * (c + 1)]
        o_ref[:, 32 * a:32 * (a + 1)] = m


def _tc_edge(ea_pp, xj_p, w1bd, b1t, w2bd, b2t, rbd):
    return pl.pallas_call(
        _edge_body,
        grid=(EP4 // BE4,),
        in_specs=[
            pl.BlockSpec((BE4, 128), lambda i: (i, 0)),
            pl.BlockSpec((BE4, 128), lambda i: (i, 0)),
            pl.BlockSpec((128, 256), lambda i: (0, 0)),
            pl.BlockSpec((1, 256), lambda i: (0, 0)),
            pl.BlockSpec((256, 4096), lambda i: (0, 0)),
            pl.BlockSpec((1, 4096), lambda i: (0, 0)),
            pl.BlockSpec((128, 4096), lambda i: (0, 0)),
        ],
        out_specs=pl.BlockSpec((BE4, 128), lambda i: (i, 0)),
        out_shape=jax.ShapeDtypeStruct((EP4, 128), jnp.float32),
    )(ea_pp, xj_p, w1bd, b1t, w2bd, b2t, rbd)


def _gru_body(p_ref, cnt_ref, h_ref, bc_ref, wih_ref, whh_ref, bih_ref,
              bhh_ref, o_ref):
    ssum = p_ref[0] + p_ref[1]
    cnt = cnt_ref[0] + cnt_ref[1]
    inv = 1.0 / jnp.maximum(cnt, 1.0)
    m = jax.nn.relu(ssum * inv + bc_ref[...])
    h = h_ref[...]
    gi = jnp.dot(m, wih_ref[...], preferred_element_type=jnp.float32) + bih_ref[...]
    gh = jnp.dot(h, whh_ref[...], preferred_element_type=jnp.float32) + bhh_ref[...]
    r = jax.nn.sigmoid(gi[:, 0:128] + gh[:, 0:128])
    z = jax.nn.sigmoid(gi[:, 128:256] + gh[:, 128:256])
    n = jnp.tanh(gi[:, 256:384] + r * gh[:, 256:384])
    o_ref[...] = (1.0 - z) * n + z * h


def _tc_gru(parts, cnt, h, bct, wihbd, whhbd, biht, bhht):
    return pl.pallas_call(
        _gru_body,
        in_specs=[
            pl.BlockSpec((2, NP4, 128), lambda: (0, 0, 0)),
            pl.BlockSpec((2, NP4, 128), lambda: (0, 0, 0)),
            pl.BlockSpec((NP4, 128), lambda: (0, 0)),
            pl.BlockSpec((1, 128), lambda: (0, 0)),
            pl.BlockSpec((128, 384), lambda: (0, 0)),
            pl.BlockSpec((128, 384), lambda: (0, 0)),
            pl.BlockSpec((1, 384), lambda: (0, 0)),
            pl.BlockSpec((1, 384), lambda: (0, 0)),
        ],
        out_specs=pl.BlockSpec((NP4, 128), lambda: (0, 0)),
        out_shape=jax.ShapeDtypeStruct((NP4, 128), jnp.float32),
    )(parts, cnt, h, bct, wihbd, whhbd, biht, bhht)


def _s2s_body(x_ref, b_ref, wih_ref, whh_ref, bih_ref, bhh_ref, o_ref):
    x = x_ref[...]
    seg = b_ref[...]
    gids = lax.broadcasted_iota(jnp.int32, (NP, NG), 1)
    m1 = (seg == gids).astype(jnp.float32)

    h = jnp.zeros((NG, DIM), jnp.float32)
    c = jnp.zeros((NG, DIM), jnp.float32)
    q_star = jnp.zeros((NG, 2 * DIM), jnp.float32)
    neg = jnp.float32(-3.0e38)

    for _ in range(STEPS):
        g = (
            jnp.dot(q_star, wih_ref[...], preferred_element_type=jnp.float32)
            + bih_ref[...]
            + jnp.dot(h, whh_ref[...], preferred_element_type=jnp.float32)
            + bhh_ref[...]
        )
        i = jax.nn.sigmoid(g[:, 0:DIM])
        f = jax.nn.sigmoid(g[:, DIM:2 * DIM])
        gc = jnp.tanh(g[:, 2 * DIM:3 * DIM])
        o = jax.nn.sigmoid(g[:, 3 * DIM:])
        c = f * c + i * gc
        h = o * jnp.tanh(c)
        q = h

        qb = jnp.dot(m1, q, preferred_element_type=jnp.float32)
        e = jnp.sum(x * qb, axis=1, keepdims=True)
        ebc = jnp.where(m1 > 0.0, e, neg)
        emax = jnp.max(ebc, axis=0, keepdims=True)
        emax = jnp.where(emax > -1.0e38, emax, 0.0)
        emax_n = jnp.dot(m1, emax.reshape(NG, 1),
                         preferred_element_type=jnp.float32)
        ex = jnp.exp(e - emax_n)
        denom = lax.dot_general(m1, ex, (((0,), (0,)), ((), ())),
                                preferred_element_type=jnp.float32)
        denom_n = jnp.dot(m1, denom, preferred_element_type=jnp.float32)
        a = ex / (denom_n + 1e-16)
        r = lax.dot_general(m1, a * x, (((0,), (0,)), ((), ())),
                            preferred_element_type=jnp.float32)
        q_star = jnp.concatenate([q, r], axis=1)

    o_ref[...] = q_star


def _tc_s2s(x, batch2, wiht, whht, bih, bhh):
    return pl.pallas_call(
        _s2s_body,
        in_specs=[
            pl.BlockSpec((NP, DIM), lambda: (0, 0)),
            pl.BlockSpec((NP, 1), lambda: (0, 0)),
            pl.BlockSpec((2 * DIM, 4 * DIM), lambda: (0, 0)),
            pl.BlockSpec((DIM, 4 * DIM), lambda: (0, 0)),
            pl.BlockSpec((1, 4 * DIM), lambda: (0, 0)),
            pl.BlockSpec((1, 4 * DIM), lambda: (0, 0)),
        ],
        out_specs=pl.BlockSpec((NG, 2 * DIM), lambda: (0, 0)),
        out_shape=jax.ShapeDtypeStruct((NG, 2 * DIM), jnp.float32),
    )(x, batch2, wiht, whht, bih, bhh)


@jax.jit
def kernel(x, edge_index, edge_attr, batch, W0, b0, We1, be1, We2, be2,
           b_conv, gru_Wih, gru_Whh, gru_bih, gru_bhh, s2s_Wih, s2s_Whh,
           s2s_bih, s2s_bhh):
    f32 = jnp.float32
    bf16 = jnp.bfloat16
    eye4 = jnp.eye(4, dtype=f32)

    xq = jnp.pad(x, ((0, NP - N), (0, 0))).reshape(NP4, 4 * NF)
    src = edge_index[0].astype(jnp.int32)
    dst = edge_index[1].astype(jnp.int32)
    src2 = jnp.zeros((EPAD,), jnp.int32).at[:E].set(src).reshape(NW, EPW)
    dst3 = jnp.full((EPAD,), NP, jnp.int32).at[:E].set(dst).reshape(NW, NCH, CHUNK)
    ea_pp = jnp.pad(
        jnp.pad(edge_attr.astype(bf16), ((0, EPAD - E), (0, 3)))
        .reshape(EP4, 4, 8),
        ((0, 0), (0, 0), (0, 24))).reshape(EP4, 128)
    batch2 = jnp.full((NP, 1), NG, jnp.int32).at[:N, 0].set(batch.astype(jnp.int32))

    w0bd = jnp.kron(eye4, W0.T)
    b0t = jnp.tile(b0, 4).reshape(1, 128)
    w1slot = jnp.zeros((DIM, 64), f32).at[:5].set(We1.T)
    w1bd = jnp.kron(eye4, w1slot).astype(bf16)
    b1t = jnp.tile(be1, 4).reshape(1, 256)
    w2t = We2.T
    w2bd = jnp.kron(eye4, w2t).astype(bf16)
    b2t = jnp.tile(be2, 4).reshape(1, 4096)
    cols = jnp.arange(DIM * DIM)
    imap = 4 * (cols // 128) + (cols % 128) // DIM
    rmat = (jnp.arange(DIM)[:, None] == imap[None, :]).astype(f32)
    rbd = jnp.kron(eye4, rmat).astype(bf16)
    bct = jnp.tile(b_conv, 4).reshape(1, 128)

    wih3 = gru_Wih.T.reshape(DIM, 3, DIM)
    whh3 = gru_Whh.T.reshape(DIM, 3, DIM)
    d4 = jnp.arange(4)
    wihbd = (jnp.zeros((4, DIM, 3, 4, DIM), f32)
             .at[d4, :, :, d4, :]
             .set(jnp.broadcast_to(wih3, (4, DIM, 3, DIM)))
             .reshape(4 * DIM, 3 * 4 * DIM))
    whhbd = (jnp.zeros((4, DIM, 3, 4, DIM), f32)
             .at[d4, :, :, d4, :]
             .set(jnp.broadcast_to(whh3, (4, DIM, 3, DIM)))
             .reshape(4 * DIM, 3 * 4 * DIM))
    biht = jnp.tile(gru_bih.reshape(3, 1, DIM), (1, 4, 1)).reshape(1, 384)
    bhht = jnp.tile(gru_bhh.reshape(3, 1, DIM), (1, 4, 1)).reshape(1, 384)

    s2s_wiht = s2s_Wih.T
    s2s_whht = s2s_Whh.T
    s2s_bihr = s2s_bih.reshape(1, 4 * DIM)
    s2s_bhhr = s2s_bhh.reshape(1, 4 * DIM)

    zeros_acc = jnp.zeros((ACC, DIM), f32)
    ones_chunk = jnp.ones((CHUNK, DIM), f32)

    out_p = _tc_proj(xq, w0bd, b0t)

    cnt_p = None
    for it in range(3):
        table = out_p.reshape(NP, DIM)
        if it == 0:
            xj, cnt = _sc_gather_count(table, src2, dst3, ones_chunk,
                                       zeros_acc)
            cnt_p = cnt.reshape(2, NP4, 128)
        else:
            xj = _sc_gather(table, src2)
        xj_p = xj.reshape(EP4, 128)
        msg_p = _tc_edge(ea_pp, xj_p, w1bd, b1t, w2bd, b2t, rbd)
        parts = _sc_scatter(msg_p.reshape(EPAD, DIM), dst3, zeros_acc)
        parts_p = parts.reshape(2, NP4, 128)
        out_p = _tc_gru(parts_p, cnt_p, out_p, bct, wihbd, whhbd, biht, bhht)

    feat = out_p.reshape(NP, DIM)
    pooled = _tc_s2s(feat, batch2, s2s_wiht, s2s_whht, s2s_bihr, s2s_bhhr)
    return (pooled, feat[:N])


# speedup vs baseline: 1.0580x; 1.0580x over previous
import functools
import jax
import jax.numpy as jnp
from jax import lax
from jax.experimental import pallas as pl
from jax.experimental.pallas import tpu as pltpu
from jax.experimental.pallas import tpu_sc as plsc

N = 10000
E = 160000
NF = 128
DIM = 32
NG = 64
STEPS = 3

NP = 10240
NP4 = NP // 4
NW = 32
CHUNK = 128
EPW = 5120
EPAD = NW * EPW
EP4 = EPAD // 4
NCH = EPW // CHUNK
ACC = 10256
ZPW = ACC // 16
OPW = NP // 16
BE4 = 1024


@functools.cache
def _mesh():
    return plsc.VectorSubcoreMesh(core_axis_name="c", subcore_axis_name="s")


SEG = 4
SGC = NCH // SEG
SGR = SGC * CHUNK


def _gather_pipeline(table, idx_v, out, buf_a, buf_b, sem_g, sem_w, base,
                     seg_hook=None):
    bufs = [buf_a, buf_b]
    writes = [None, None]
    for g in range(SEG):
        buf = bufs[g % 2]
        if writes[g % 2] is not None:
            writes[g % 2].wait()
        cp = pltpu.async_copy(
            table.at[idx_v.at[pl.ds(g * SGR, SGR)]], buf, sem_g
        )
        if seg_hook is not None:
            seg_hook(g)
        cp.wait()
        writes[g % 2] = pltpu.async_copy(
            buf, out.at[pl.ds(base + g * SGR, SGR)], sem_w
        )
    writes[0].wait()
    writes[1].wait()


def _gather_body(table, idx3, out, idx_v, buf_a, buf_b, sem_g, sem_w):
    c = lax.axis_index("c")
    s = lax.axis_index("s")
    wid = s * 2 + c
    pltpu.sync_copy(idx3.at[wid], idx_v)
    _gather_pipeline(table, idx_v, out, buf_a, buf_b, sem_g, sem_w, wid * EPW)


def _sc_gather(table, idx3):
    return pl.kernel(
        _gather_body,
        out_type=jax.ShapeDtypeStruct((EPAD, DIM), jnp.float32),
        mesh=_mesh(),
        compiler_params=pltpu.CompilerParams(use_tc_tiling_on_sc=False),
        scratch_types=[
            pltpu.VMEM((EPW,), jnp.int32),
            pltpu.VMEM((SGR, DIM), jnp.float32),
            pltpu.VMEM((SGR, DIM), jnp.float32),
            pltpu.SemaphoreType.DMA,
            pltpu.SemaphoreType.DMA,
        ],
    )(table, idx3)


def _gather_count_body(table, idx3, cidx3, ones, zeros, out, cnt_out,
                       acc, idx_v, cidx_v, ones_v, buf_a, buf_b,
                       sem_g, sem_w):
    c = lax.axis_index("c")
    s = lax.axis_index("s")
    wid = s * 2 + c
    pltpu.sync_copy(zeros.at[pl.ds(s * ZPW, ZPW)], acc.at[pl.ds(s * ZPW, ZPW)])
    pltpu.sync_copy(idx3.at[wid], idx_v)
    pltpu.sync_copy(cidx3.at[wid], cidx_v)
    pltpu.sync_copy(ones, ones_v)
    plsc.subcore_barrier()

    def count_hook(g):
        def body(j, carry):
            pltpu.sync_copy(ones_v, acc.at[cidx_v.at[g * SGC + j]], add=True)
            return carry
        lax.fori_loop(0, SGC, body, 0, unroll=False)

    _gather_pipeline(table, idx_v, out, buf_a, buf_b, sem_g, sem_w,
                     wid * EPW, seg_hook=count_hook)
    plsc.subcore_barrier()
    pltpu.sync_copy(acc.at[pl.ds(s * OPW, OPW)],
                    cnt_out.at[c].at[pl.ds(s * OPW, OPW)])


def _sc_gather_count(table, idx3, cidx3, ones, zeros):
    return pl.kernel(
        _gather_count_body,
        out_type=(
            jax.ShapeDtypeStruct((EPAD, DIM), jnp.float32),
            jax.ShapeDtypeStruct((2, NP, DIM), jnp.float32),
        ),
        mesh=_mesh(),
        compiler_params=pltpu.CompilerParams(use_tc_tiling_on_sc=False),
        scratch_types=[
            pltpu.VMEM_SHARED((ACC, DIM), jnp.float32),
            pltpu.VMEM((EPW,), jnp.int32),
            pltpu.VMEM((NCH, CHUNK), jnp.int32),
            pltpu.VMEM((CHUNK, DIM), jnp.float32),
            pltpu.VMEM((SGR, DIM), jnp.float32),
            pltpu.VMEM((SGR, DIM), jnp.float32),
            pltpu.SemaphoreType.DMA,
            pltpu.SemaphoreType.DMA,
        ],
    )(table, idx3, cidx3, ones, zeros)


def _scatter_body(msg, idx3, zeros, out, acc, idx_v, buf_v):
    c = lax.axis_index("c")
    s = lax.axis_index("s")
    wid = s * 2 + c
    pltpu.sync_copy(zeros.at[pl.ds(s * ZPW, ZPW)], acc.at[pl.ds(s * ZPW, ZPW)])
    pltpu.sync_copy(idx3.at[wid], idx_v)
    plsc.subcore_barrier()

    def body(j, carry):
        pltpu.sync_copy(msg.at[pl.ds(wid * EPW + j * CHUNK, CHUNK)], buf_v)
        pltpu.sync_copy(buf_v, acc.at[idx_v.at[j]], add=True)
        return carry

    lax.fori_loop(0, NCH, body, 0, unroll=False)
    plsc.subcore_barrier()
    pltpu.sync_copy(acc.at[pl.ds(s * OPW, OPW)], out.at[c].at[pl.ds(s * OPW, OPW)])


def _sc_scatter(msg, idx3, zeros):
    return pl.kernel(
        _scatter_body,
        out_type=jax.ShapeDtypeStruct((2, NP, DIM), jnp.float32),
        mesh=_mesh(),
        compiler_params=pltpu.CompilerParams(use_tc_tiling_on_sc=False),
        scratch_types=[
            pltpu.VMEM_SHARED((ACC, DIM), jnp.float32),
            pltpu.VMEM((NCH, CHUNK), jnp.int32),
            pltpu.VMEM((CHUNK, DIM), jnp.float32),
        ],
    )(msg, idx3, zeros)


def _proj_body(x_ref, w_ref, b_ref, o_ref):
    o_ref[...] = jax.nn.relu(
        jnp.dot(x_ref[...], w_ref[...], preferred_element_type=jnp.float32)
        + b_ref[...]
    )


def _tc_proj(xq, w0bd, b0t):
    return pl.pallas_call(
        _proj_body,
        in_specs=[
            pl.BlockSpec((NP4, 4 * NF), lambda: (0, 0)),
            pl.BlockSpec((4 * NF, 128), lambda: (0, 0)),
            pl.BlockSpec((1, 128), lambda: (0, 0)),
        ],
        out_specs=pl.BlockSpec((NP4, 128), lambda: (0, 0)),
        out_shape=jax.ShapeDtypeStruct((NP4, 128), jnp.float32),
    )(xq, w0bd, b0t)


def _edge_body(ea_ref, xj_ref, w1_ref, b1_ref, w2_ref, b2_ref, r_ref, o_ref):
    hidden = jax.nn.relu(
        jnp.dot(ea_ref[...], w1_ref[...], preferred_element_type=jnp.float32)
        + b1_ref[...]
    ).astype(jnp.bfloat16)
    theta = (
        jnp.dot(hidden, w2_ref[...], preferred_element_type=jnp.float32)
        + b2_ref[...]
    )
    xjr = jnp.dot(xj_ref[...].astype(jnp.bfloat16), r_ref[...],
                  preferred_element_type=jnp.float32)
    prod = theta * xjr
    for a in range(4):
        s1 = prod[:, 1024 * a:1024 * a + 128]
        for q in range(1, 8):
            s1 = s1 + prod[:, 1024 * a + 128 * q:1024 * a + 128 * (q + 1)]
        m = s1[:, 0:DIM]
        for c in range(1, 4):
            m = m + s1[:, DIM * c:DIM * (c + 1)]
        o_ref[:, 32 * a:32 * (a + 1)] = m


def _tc_edge(ea_pp, xj_p, w1bd, b1t, w2bd, b2t, rbd):
    return pl.pallas_call(
        _edge_body,
        grid=(EP4 // BE4,),
        in_specs=[
            pl.BlockSpec((BE4, 128), lambda i: (i, 0)),
            pl.BlockSpec((BE4, 128), lambda i: (i, 0)),
            pl.BlockSpec((128, 256), lambda i: (0, 0)),
            pl.BlockSpec((1, 256), lambda i: (0, 0)),
            pl.BlockSpec((256, 4096), lambda i: (0, 0)),
            pl.BlockSpec((1, 4096), lambda i: (0, 0)),
            pl.BlockSpec((128, 4096), lambda i: (0, 0)),
        ],
        out_specs=pl.BlockSpec((BE4, 128), lambda i: (i, 0)),
        out_shape=jax.ShapeDtypeStruct((EP4, 128), jnp.float32),
    )(ea_pp, xj_p, w1bd, b1t, w2bd, b2t, rbd)


def _gru_body(p_ref, cnt_ref, h_ref, bc_ref, wih_ref, whh_ref, bih_ref,
              bhh_ref, o_ref):
    ssum = p_ref[0] + p_ref[1]
    cnt = cnt_ref[0] + cnt_ref[1]
    inv = 1.0 / jnp.maximum(cnt, 1.0)
    m = jax.nn.relu(ssum * inv + bc_ref[...])
    h = h_ref[...]
    gi = jnp.dot(m, wih_ref[...], preferred_element_type=jnp.float32) + bih_ref[...]
    gh = jnp.dot(h, whh_ref[...], preferred_element_type=jnp.float32) + bhh_ref[...]
    r = jax.nn.sigmoid(gi[:, 0:128] + gh[:, 0:128])
    z = jax.nn.sigmoid(gi[:, 128:256] + gh[:, 128:256])
    n = jnp.tanh(gi[:, 256:384] + r * gh[:, 256:384])
    o_ref[...] = (1.0 - z) * n + z * h


def _tc_gru(parts, cnt, h, bct, wihbd, whhbd, biht, bhht):
    return pl.pallas_call(
        _gru_body,
        in_specs=[
            pl.BlockSpec((2, NP4, 128), lambda: (0, 0, 0)),
            pl.BlockSpec((2, NP4, 128), lambda: (0, 0, 0)),
            pl.BlockSpec((NP4, 128), lambda: (0, 0)),
            pl.BlockSpec((1, 128), lambda: (0, 0)),
            pl.BlockSpec((128, 384), lambda: (0, 0)),
            pl.BlockSpec((128, 384), lambda: (0, 0)),
            pl.BlockSpec((1, 384), lambda: (0, 0)),
            pl.BlockSpec((1, 384), lambda: (0, 0)),
        ],
        out_specs=pl.BlockSpec((NP4, 128), lambda: (0, 0)),
        out_shape=jax.ShapeDtypeStruct((NP4, 128), jnp.float32),
    )(parts, cnt, h, bct, wihbd, whhbd, biht, bhht)


def _s2s_body(x_ref, b_ref, wih_ref, whh_ref, bih_ref, bhh_ref, o_ref):
    x = x_ref[...]
    seg = b_ref[...]
    gids = lax.broadcasted_iota(jnp.int32, (NP, NG), 1)
    m1 = (seg == gids).astype(jnp.float32)

    h = jnp.zeros((NG, DIM), jnp.float32)
    c = jnp.zeros((NG, DIM), jnp.float32)
    q_star = jnp.zeros((NG, 2 * DIM), jnp.float32)
    neg = jnp.float32(-3.0e38)

    for _ in range(STEPS):
        g = (
            jnp.dot(q_star, wih_ref[...], preferred_element_type=jnp.float32)
            + bih_ref[...]
            + jnp.dot(h, whh_ref[...], preferred_element_type=jnp.float32)
            + bhh_ref[...]
        )
        i = jax.nn.sigmoid(g[:, 0:DIM])
        f = jax.nn.sigmoid(g[:, DIM:2 * DIM])
        gc = jnp.tanh(g[:, 2 * DIM:3 * DIM])
        o = jax.nn.sigmoid(g[:, 3 * DIM:])
        c = f * c + i * gc
        h = o * jnp.tanh(c)
        q = h

        qb = jnp.dot(m1, q, preferred_element_type=jnp.float32)
        e = jnp.sum(x * qb, axis=1, keepdims=True)
        ebc = jnp.where(m1 > 0.0, e, neg)
        emax = jnp.max(ebc, axis=0, keepdims=True)
        emax = jnp.where(emax > -1.0e38, emax, 0.0)
        emax_n = jnp.dot(m1, emax.reshape(NG, 1),
                         preferred_element_type=jnp.float32)
        ex = jnp.exp(e - emax_n)
        denom = lax.dot_general(m1, ex, (((0,), (0,)), ((), ())),
                                preferred_element_type=jnp.float32)
        denom_n = jnp.dot(m1, denom, preferred_element_type=jnp.float32)
        a = ex / (denom_n + 1e-16)
        r = lax.dot_general(m1, a * x, (((0,), (0,)), ((), ())),
                            preferred_element_type=jnp.float32)
        q_star = jnp.concatenate([q, r], axis=1)

    o_ref[...] = q_star


def _tc_s2s(x, batch2, wiht, whht, bih, bhh):
    return pl.pallas_call(
        _s2s_body,
        in_specs=[
            pl.BlockSpec((NP, DIM), lambda: (0, 0)),
            pl.BlockSpec((NP, 1), lambda: (0, 0)),
            pl.BlockSpec((2 * DIM, 4 * DIM), lambda: (0, 0)),
            pl.BlockSpec((DIM, 4 * DIM), lambda: (0, 0)),
            pl.BlockSpec((1, 4 * DIM), lambda: (0, 0)),
            pl.BlockSpec((1, 4 * DIM), lambda: (0, 0)),
        ],
        out_specs=pl.BlockSpec((NG, 2 * DIM), lambda: (0, 0)),
        out_shape=jax.ShapeDtypeStruct((NG, 2 * DIM), jnp.float32),
    )(x, batch2, wiht, whht, bih, bhh)


@jax.jit
def kernel(x, edge_index, edge_attr, batch, W0, b0, We1, be1, We2, be2,
           b_conv, gru_Wih, gru_Whh, gru_bih, gru_bhh, s2s_Wih, s2s_Whh,
           s2s_bih, s2s_bhh):
    f32 = jnp.float32
    bf16 = jnp.bfloat16
    eye4 = jnp.eye(4, dtype=f32)

    xq = jnp.zeros((NP, NF), f32).at[:N].set(x).reshape(NP4, 4 * NF)
    src = edge_index[0].astype(jnp.int32)
    dst = edge_index[1].astype(jnp.int32)
    src2 = jnp.zeros((EPAD,), jnp.int32).at[:E].set(src).reshape(NW, EPW)
    dst3 = jnp.full((EPAD,), NP, jnp.int32).at[:E].set(dst).reshape(NW, NCH, CHUNK)
    ea_pp = jnp.zeros((EPAD, DIM), bf16).at[:E, :5].set(
        edge_attr.astype(bf16)).reshape(EP4, 128)
    batch2 = jnp.full((NP, 1), NG, jnp.int32).at[:N, 0].set(batch.astype(jnp.int32))

    w0bd = jnp.kron(eye4, W0.T)
    b0t = jnp.tile(b0, 4).reshape(1, 128)
    w1slot = jnp.zeros((DIM, 64), f32).at[:5].set(We1.T)
    w1bd = jnp.kron(eye4, w1slot).astype(bf16)
    b1t = jnp.tile(be1, 4).reshape(1, 256)
    w2t = We2.T
    w2bd = jnp.kron(eye4, w2t).astype(bf16)
    b2t = jnp.tile(be2, 4).reshape(1, 4096)
    cols = jnp.arange(DIM * DIM)
    imap = 4 * (cols // 128) + (cols % 128) // DIM
    rmat = (jnp.arange(DIM)[:, None] == imap[None, :]).astype(f32)
    rbd = jnp.kron(eye4, rmat).astype(bf16)
    bct = jnp.tile(b_conv, 4).reshape(1, 128)

    wih3 = gru_Wih.T.reshape(DIM, 3, DIM)
    whh3 = gru_Whh.T.reshape(DIM, 3, DIM)
    d4 = jnp.arange(4)
    wihbd = (jnp.zeros((4, DIM, 3, 4, DIM), f32)
             .at[d4, :, :, d4, :]
             .set(jnp.broadcast_to(wih3, (4, DIM, 3, DIM)))
             .reshape(4 * DIM, 3 * 4 * DIM))
    whhbd = (jnp.zeros((4, DIM, 3, 4, DIM), f32)
             .at[d4, :, :, d4, :]
             .set(jnp.broadcast_to(whh3, (4, DIM, 3, DIM)))
             .reshape(4 * DIM, 3 * 4 * DIM))
    biht = jnp.tile(gru_bih.reshape(3, 1, DIM), (1, 4, 1)).reshape(1, 384)
    bhht = jnp.tile(gru_bhh.reshape(3, 1, DIM), (1, 4, 1)).reshape(1, 384)

    s2s_wiht = s2s_Wih.T
    s2s_whht = s2s_Whh.T
    s2s_bihr = s2s_bih.reshape(1, 4 * DIM)
    s2s_bhhr = s2s_bhh.reshape(1, 4 * DIM)

    zeros_acc = jnp.zeros((ACC, DIM), f32)
    ones_chunk = jnp.ones((CHUNK, DIM), f32)

    out_p = _tc_proj(xq, w0bd, b0t)

    cnt_p = None
    for it in range(3):
        table = out_p.reshape(NP, DIM)
        if it == 0:
            xj, cnt = _sc_gather_count(table, src2, dst3, ones_chunk,
                                       zeros_acc)
            cnt_p = cnt.reshape(2, NP4, 128)
        else:
            xj = _sc_gather(table, src2)
        xj_p = xj.reshape(EP4, 128)
        msg_p = _tc_edge(ea_pp, xj_p, w1bd, b1t, w2bd, b2t, rbd)
        parts = _sc_scatter(msg_p.reshape(EPAD, DIM), dst3, zeros_acc)
        parts_p = parts.reshape(2, NP4, 128)
        out_p = _tc_gru(parts_p, cnt_p, out_p, bct, wihbd, whhbd, biht, bhht)

    feat = out_p.reshape(NP, DIM)
    pooled = _tc_s2s(feat, batch2, s2s_wiht, s2s_whht, s2s_bihr, s2s_bhhr)
    return (pooled, feat[:N])
